# Initial kernel scaffold; baseline (speedup 1.0000x reference)
import jax
import jax.numpy as jnp
from jax.experimental import pallas as pl


def kernel(inputs, emb_table, W, b):
    raise NotImplementedError("write your pallas kernel here")


# speedup vs baseline: 22.0174x; 22.0174x over previous
import functools

import jax
import jax.numpy as jnp
from jax import lax
from jax.experimental import pallas as pl
from jax.experimental.pallas import tpu as pltpu
from jax.experimental.pallas import tpu_sc as plsc

_CPAD = 16
_NWORKERS = 32
_NBUF = 8


def _project_body(emb_ref, w_ref, p_ref):
    p_ref[...] = lax.dot_general(
        emb_ref[...], w_ref[...],
        (((1,), (1,)), ((), ())),
        preferred_element_type=jnp.float32,
    )


def _project(emb, w_pad):
    v, d = emb.shape
    br = 5000
    return pl.pallas_call(
        _project_body,
        grid=(v // br,),
        in_specs=[
            pl.BlockSpec((br, d), lambda i: (i, 0)),
            pl.BlockSpec((_CPAD, d), lambda i: (0, 0)),
        ],
        out_specs=pl.BlockSpec((br, _CPAD), lambda i: (i, 0)),
        out_shape=jax.ShapeDtypeStruct((v, _CPAD), jnp.float32),
    )(emb, w_pad)


def _gather_sum(p, idx3):
    _, s_dim, k_dim = idx3.shape
    mesh = plsc.VectorSubcoreMesh(core_axis_name="c", subcore_axis_name="s")

    @functools.partial(
        pl.kernel,
        out_type=jax.ShapeDtypeStruct((_NWORKERS, s_dim, _CPAD), jnp.float32),
        mesh=mesh,
        scratch_types=[
            pltpu.VMEM((s_dim, k_dim), jnp.int32),
            pltpu.VMEM((_NBUF, k_dim, _CPAD), jnp.float32),
            pltpu.VMEM((s_dim, _CPAD), jnp.float32),
            pltpu.SemaphoreType.DMA((_NBUF,)),
        ],
        compiler_params=pltpu.CompilerParams(use_tc_tiling_on_sc=False),
    )
    def k(p_hbm, idx_hbm, out_hbm, idx_v, rows_v, acc_v, gsems):
        cid = lax.axis_index("c")
        sid = lax.axis_index("s")
        w = cid * 16 + sid

        pltpu.sync_copy(idx_hbm.at[w], idx_v)

        for b in range(_NBUF):
            pltpu.async_copy(p_hbm.at[idx_v.at[b]], rows_v.at[b], gsems.at[b])

        def outer(g, carry):
            for b in range(_NBUF):
                s = g * _NBUF + b
                rows = rows_v.at[b]
                pltpu.make_async_copy(
                    p_hbm.at[idx_v.at[s]], rows, gsems.at[b]).wait()

                def red(i, accs):
                    a0, a1, a2, a3 = accs
                    i4 = i * 4
                    return (a0 + rows[i4], a1 + rows[i4 + 1],
                            a2 + rows[i4 + 2], a3 + rows[i4 + 3])

                z = jnp.zeros((_CPAD,), jnp.float32)
                a0, a1, a2, a3 = lax.fori_loop(0, k_dim // 4, red, (z, z, z, z))
                acc_v[s] = (a0 + a1) + (a2 + a3)

                nxt = s + _NBUF

                @pl.when(nxt < s_dim)
                def _():
                    pltpu.async_copy(
                        p_hbm.at[idx_v.at[nxt]], rows, gsems.at[b])

            return carry

        lax.fori_loop(0, s_dim // _NBUF, outer, 0)
        pltpu.sync_copy(acc_v, out_hbm.at[w])

    return k(p, idx3)


def kernel(inputs, emb_table, W, b):
    batch, s_dim = inputs.shape
    ncls, d = W.shape
    k_dim = batch // _NWORKERS

    w_pad = jnp.zeros((_CPAD, d), jnp.float32).at[:ncls].set(W) * (1.0 / batch)
    p = _project(emb_table, w_pad)

    idx3 = inputs.T.reshape(s_dim, _NWORKERS, k_dim).swapaxes(0, 1)

    partials = _gather_sum(p, idx3)
    return partials.sum(axis=0)[:, :ncls] + b[None, :]
